# baseline (speedup 1.0000x reference)
import jax
import jax.numpy as jnp
from jax.experimental import pallas as pl
from jax.experimental.pallas import tpu as pltpu
from jax.experimental.pallas import tpu_sc as plsc

_N = 8192
_P = _N + 1
_TOTAL = _N * _N
_NC, _NS = 2, 16
_NW = _NC * _NS
_PER_W = _TOTAL // _NW
_CHUNK = 8 * _N
_NCH = _PER_W // _CHUNK
_MAXOFF = 8 * (_NCH - 1)
_WIN = _CHUNK + _MAXOFF
_BLEN = _WIN + 8
_C0 = _PER_W % _P


def _sc_body(out_ref, b_ref, sem):
    c = jax.lax.axis_index("c")
    s = jax.lax.axis_index("s")
    w = s * _NC + c

    zeros16 = jnp.zeros((16,), jnp.float32)

    def _zero(it, carry):
        base = it * 256
        for u in range(16):
            b_ref[pl.ds(base + u * 16, 16)] = zeros16
        return carry

    jax.lax.fori_loop(0, _BLEN // 256, _zero, 0)

    cw = jax.lax.rem(w * _C0, _P)
    phi = cw - _MAXOFF
    m = jax.lax.rem(phi, _P)
    i0 = jax.lax.rem(_P - m, _P)
    lane_iota = jax.lax.iota(jnp.int32, 16)
    for k in range(_BLEN // _P + 1):
        idx = i0 + k * _P

        @pl.when(idx < _WIN)
        def _patch():
            base16 = (idx // 16) * 16
            d = lane_iota - (idx - base16)
            onehot = jnp.maximum(1 - d * d, 0)
            b_ref[pl.ds(base16, 16)] = onehot.astype(jnp.float32)

    copies = []
    for t in range(_NCH):
        copies.append(pltpu.async_copy(
            b_ref.at[pl.ds(_MAXOFF - 8 * t, _CHUNK)],
            out_ref.at[pl.ds(w * _PER_W + t * _CHUNK, _CHUNK)],
            sem,
        ))
    for cp in copies:
        cp.wait()


_sc_call = pl.kernel(
    _sc_body,
    out_type=jax.ShapeDtypeStruct((_TOTAL,), jnp.float32),
    mesh=plsc.VectorSubcoreMesh(core_axis_name="c", subcore_axis_name="s"),
    scratch_types=[
        pltpu.VMEM((_BLEN,), jnp.float32),
        pltpu.SemaphoreType.DMA,
    ],
)


def kernel(x):
    seq_len = x.shape[1]
    flat = _sc_call()
    return flat.reshape(seq_len, _N).astype(x.dtype)


# speedup vs baseline: 3.7421x; 3.7421x over previous
import jax
import jax.numpy as jnp
from jax.experimental import pallas as pl
from jax.experimental.pallas import tpu as pltpu
from jax.experimental.pallas import tpu_sc as plsc

_N = 8192
_NC, _NS = 2, 16
_NW = _NC * _NS
_ROWS_W = _N // _NW
_K = 4
_NCH = _ROWS_W // _K
_NBUF = 2


def _onehot16(lane):
    d = jax.lax.iota(jnp.int32, 16) - lane
    return jnp.maximum(1 - d * d, 0).astype(jnp.float32)


def _sc_body(out_ref, ba_ref, bb_ref, sema, semb):
    c = jax.lax.axis_index("c")
    s = jax.lax.axis_index("s")
    w = s * _NC + c
    r0 = w * _ROWS_W

    bufs = (ba_ref, bb_ref)
    sems = (sema, semb)

    zeros16 = jnp.zeros((16,), jnp.float32)

    def _zero_rows(it, carry):
        base = it * 128
        for u in range(8):
            off = base + u * 16
            for b in bufs:
                for r in range(_K):
                    b[r, pl.ds(off, 16)] = zeros16
        return carry

    jax.lax.fori_loop(0, _N // 128, _zero_rows, 0)

    copies = [None] * _NCH
    for t in range(_NCH):
        buf = bufs[t % _NBUF]
        sem = sems[t % _NBUF]
        if t >= _NBUF:
            copies[t - _NBUF].wait()
        for i in range(_K):
            col = _K * t + i
            if t >= _NBUF:
                pcol = col - _K * _NBUF
                b16 = (pcol // 16) * 16
                buf[i, pl.ds(r0 + b16, 16)] = zeros16
            b16 = (col // 16) * 16
            buf[i, pl.ds(r0 + b16, 16)] = _onehot16(col - b16)
        copies[t] = pltpu.async_copy(
            buf, out_ref.at[pl.ds(r0 + _K * t, _K), :], sem)
    for t in range(_NCH - _NBUF, _NCH):
        copies[t].wait()


_sc_call = pl.kernel(
    _sc_body,
    out_type=jax.ShapeDtypeStruct((_N, _N), jnp.float32),
    mesh=plsc.VectorSubcoreMesh(core_axis_name="c", subcore_axis_name="s"),
    scratch_types=[
        pltpu.VMEM((_K, _N), jnp.float32),
        pltpu.VMEM((_K, _N), jnp.float32),
        pltpu.SemaphoreType.DMA,
        pltpu.SemaphoreType.DMA,
    ],
)


def kernel(x):
    del x
    return _sc_call()


# speedup vs baseline: 4.6046x; 1.2305x over previous
import jax
import jax.numpy as jnp
from jax.experimental import pallas as pl
from jax.experimental.pallas import tpu as pltpu

_N = 8192
_T = 512
_G = _N // _T
_DEPTH = 16


def _body(o_ref, z_ref, e_ref, sems):
    i = pl.program_id(0)
    j = jax.lax.rem(i + 1 + pl.program_id(1), _G)
    step = i * _G + pl.program_id(1)

    @pl.when(step == 0)
    def _init_z():
        z_ref[...] = jnp.zeros((_T, _T), jnp.float32)

    @pl.when(step == 1)
    def _init_e():
        r = jax.lax.broadcasted_iota(jnp.int32, (_T, _T), 0)
        c = jax.lax.broadcasted_iota(jnp.int32, (_T, _T), 1)
        e_ref[...] = (r == c).astype(jnp.float32)

    slot = jax.lax.rem(step, _DEPTH)
    dst = o_ref.at[pl.ds(i * _T, _T), pl.ds(j * _T, _T)]

    @pl.when(step >= _DEPTH)
    def _drain_one():
        pltpu.make_async_copy(z_ref, dst, sems.at[slot]).wait()

    @pl.when(i == j)
    def _fire_eye():
        pltpu.make_async_copy(e_ref, dst, sems.at[slot]).start()

    @pl.when(i != j)
    def _fire_zero():
        pltpu.make_async_copy(z_ref, dst, sems.at[slot]).start()

    @pl.when(step == _G * _G - 1)
    def _drain_all():
        for k in range(_DEPTH):
            pltpu.make_async_copy(z_ref, dst, sems.at[k]).wait()


def kernel(x):
    seq_len = x.shape[1]
    return pl.pallas_call(
        _body,
        grid=(seq_len // _T, _N // _T),
        out_specs=pl.BlockSpec(memory_space=pl.ANY),
        out_shape=jax.ShapeDtypeStruct((seq_len, _N), x.dtype),
        scratch_shapes=[
            pltpu.VMEM((_T, _T), jnp.float32),
            pltpu.VMEM((_T, _T), jnp.float32),
            pltpu.SemaphoreType.DMA((_DEPTH,)),
        ],
    )()


# speedup vs baseline: 4.8494x; 1.0532x over previous
import jax
import jax.numpy as jnp
from jax.experimental import pallas as pl
from jax.experimental.pallas import tpu as pltpu

_N = 8192
_T = 512
_G = _N // _T
_DEPTH = 4


def _body(o_ref, z_ref, e_ref, sems):
    i = pl.program_id(0)
    j = jax.lax.rem(i + 1 + pl.program_id(1), _G)
    step = i * _G + pl.program_id(1)

    @pl.when(step == 0)
    def _init_z():
        z_ref[...] = jnp.zeros((_T, _T), jnp.float32)

    @pl.when(step == 1)
    def _init_e():
        r = jax.lax.broadcasted_iota(jnp.int32, (_T, _T), 0)
        c = jax.lax.broadcasted_iota(jnp.int32, (_T, _T), 1)
        e_ref[...] = (r == c).astype(jnp.float32)

    slot = jax.lax.rem(step, _DEPTH)
    dst = o_ref.at[pl.ds(i * _T, _T), pl.ds(j * _T, _T)]

    @pl.when(step >= _DEPTH)
    def _drain_one():
        pltpu.make_async_copy(z_ref, dst, sems.at[slot]).wait()

    @pl.when(i == j)
    def _fire_eye():
        pltpu.make_async_copy(e_ref, dst, sems.at[slot]).start()

    @pl.when(i != j)
    def _fire_zero():
        pltpu.make_async_copy(z_ref, dst, sems.at[slot]).start()

    @pl.when(step == _G * _G - 1)
    def _drain_all():
        for k in range(_DEPTH):
            pltpu.make_async_copy(z_ref, dst, sems.at[k]).wait()


def kernel(x):
    seq_len = x.shape[1]
    return pl.pallas_call(
        _body,
        grid=(seq_len // _T, _N // _T),
        out_specs=pl.BlockSpec(memory_space=pl.ANY),
        out_shape=jax.ShapeDtypeStruct((seq_len, _N), x.dtype),
        scratch_shapes=[
            pltpu.VMEM((_T, _T), jnp.float32),
            pltpu.VMEM((_T, _T), jnp.float32),
            pltpu.SemaphoreType.DMA((_DEPTH,)),
        ],
    )()
